# baseline (speedup 1.0000x reference)
import functools

import jax
import jax.numpy as jnp
from jax import lax
from jax.experimental import pallas as pl
from jax.experimental.pallas import tpu as pltpu
from jax.experimental.pallas import tpu_sc as plsc

VOCAB = 1000000
D = 64
BATCH = 16384
HIST = 50
NW = 32
CHUNK = 128
NBLK = HIST * (BATCH // CHUNK)
PER_W = NBLK // NW
BLK_BYTES = 8 * 8 * 128 * 4


def _process(idx_v, table_hbm, out_hbm, bufs, stages, sem_g, sem_s, rows16,
             jj, s, wid):
    k = wid * PER_W + jj
    h = k // CHUNK
    c = lax.rem(k, CHUNK)
    buf, stage = bufs[s], stages[s]
    pltpu.make_async_copy(table_hbm.at[idx_v.at[jj]], buf, sem_g[s]).wait()

    @pl.when(jj >= 2)
    def _():
        k2 = k - 2
        for tr in range(8):
            pltpu.make_async_copy(
                stage.at[pl.ds(8 * tr, 8), :],
                out_hbm.at[k2 // CHUNK, tr, lax.rem(k2, CHUNK)],
                sem_s[s],
            ).wait()

    iota, perms = rows16

    def tloop(g, _):
        rows = iota + 16 * g
        for dblk in range(4):
            for i in range(16):
                col = perms[i] + 16 * dblk
                v = plsc.load_gather(buf, [rows, col])
                plsc.store_scatter(stage, [col, rows], v)
        return 0

    lax.fori_loop(0, 8, tloop, 0)
    for tr in range(8):
        pltpu.async_copy(
            stage.at[pl.ds(8 * tr, 8), :], out_hbm.at[h, tr, c], sem_s[s]
        )


def _gather_body(idx_hbm, table_hbm, out_hbm, idx_v, buf_a, buf_b,
                 stage_a, stage_b, sem_ga, sem_gb, sem_sa, sem_sb):
    wid = lax.axis_index("s") * 2 + lax.axis_index("c")
    pltpu.sync_copy(idx_hbm.at[wid], idx_v)

    bufs = (buf_a, buf_b)
    stages = (stage_a, stage_b)
    sem_g = (sem_ga, sem_gb)
    sem_s = (sem_sa, sem_sb)
    iota = lax.iota(jnp.int32, 16)
    perms = [(iota + i) & 15 for i in range(16)]
    rows16 = (iota, perms)

    def start(j, slot):
        pltpu.async_copy(table_hbm.at[idx_v.at[j]], bufs[slot], sem_g[slot])

    start(0, 0)
    start(1, 1)

    def loop(g, _):
        j = 2 * g
        for b in range(2):
            _process(idx_v, table_hbm, out_hbm, bufs, stages, sem_g, sem_s,
                     rows16, j + b, b, wid)
            start(j + b + 2, b)
        return 0

    lax.fori_loop(0, PER_W // 2 - 1, loop, 0)
    for b in range(2):
        _process(idx_v, table_hbm, out_hbm, bufs, stages, sem_g, sem_s,
                 rows16, PER_W - 2 + b, b, wid)
    for b in range(2):
        k2 = wid * PER_W + PER_W - 2 + b
        for tr in range(8):
            pltpu.make_async_copy(
                stages[b].at[pl.ds(8 * tr, 8), :],
                out_hbm.at[k2 // CHUNK, tr, lax.rem(k2, CHUNK)],
                sem_s[b],
            ).wait()


@jax.jit
def _gather(idx_grouped, emb_table):
    mesh = plsc.VectorSubcoreMesh(core_axis_name="c", subcore_axis_name="s")
    kfn = functools.partial(
        pl.kernel,
        mesh=mesh,
        out_type=jax.ShapeDtypeStruct((HIST, 8, 128, 8, 128), jnp.float32),
        scratch_types=(
            [pltpu.VMEM((PER_W, CHUNK), jnp.int32)]
            + [pltpu.VMEM((CHUNK, D), jnp.float32) for _ in range(2)]
            + [pltpu.VMEM((D, CHUNK), jnp.float32) for _ in range(2)]
            + [pltpu.SemaphoreType.DMA for _ in range(4)]
        ),
        compiler_params=pltpu.CompilerParams(
            use_tc_tiling_on_sc=False, needs_layout_passes=False
        ),
    )(_gather_body)
    return kfn(idx_grouped, emb_table)


def kernel(indices, emb_table):
    idx_grouped = indices.T.reshape(NW, PER_W, CHUNK).astype(jnp.int32)
    out5 = _gather(idx_grouped, emb_table)
    return out5.transpose(2, 4, 0, 1, 3).reshape(BATCH, HIST, D)


# speedup vs baseline: 1.4339x; 1.4339x over previous
import functools

import jax
import jax.numpy as jnp
from jax import lax
from jax.experimental import pallas as pl
from jax.experimental.pallas import tpu as pltpu
from jax.experimental.pallas import tpu_sc as plsc

VOCAB = 1000000
D = 64
BATCH = 16384
HIST = 50
NW = 32
CHUNK = 128
NBLK = HIST * (BATCH // CHUNK)
PER_W = NBLK // NW
BLK_BYTES = 8 * 8 * 128 * 4


def _process(idx_v, table_hbm, out_hbm, bufs, stages, sem_g, sem_s, rows16,
             jj, s, wid):
    k = wid * PER_W + jj
    h = k // CHUNK
    c = lax.rem(k, CHUNK)
    buf, stage = bufs[s], stages[s]
    pltpu.make_async_copy(table_hbm.at[idx_v.at[jj]], buf, sem_g[s]).wait()

    @pl.when(jj >= 2)
    def _():
        k2 = k - 2
        for tr in range(8):
            pltpu.make_async_copy(
                stage.at[pl.ds(8 * tr, 8), :],
                out_hbm.at[k2 // CHUNK, tr, lax.rem(k2, CHUNK)],
                sem_s[s],
            ).wait()

    iota, perms = rows16

    def tloop(g, _):
        rows = iota + 16 * g
        for dblk in range(4):
            cols = [perms[i] + 16 * dblk for i in range(16)]
            vals = [plsc.load_gather(buf, [rows, col]) for col in cols]
            for col, v in zip(cols, vals):
                plsc.store_scatter(stage, [col, rows], v)
        return 0

    lax.fori_loop(0, 8, tloop, 0)
    for tr in range(8):
        pltpu.async_copy(
            stage.at[pl.ds(8 * tr, 8), :], out_hbm.at[h, tr, c], sem_s[s]
        )


def _gather_body(idx_hbm, table_hbm, out_hbm, idx_v, buf_a, buf_b,
                 stage_a, stage_b, sem_ga, sem_gb, sem_sa, sem_sb):
    wid = lax.axis_index("s") * 2 + lax.axis_index("c")
    pltpu.sync_copy(idx_hbm.at[wid], idx_v)

    bufs = (buf_a, buf_b)
    stages = (stage_a, stage_b)
    sem_g = (sem_ga, sem_gb)
    sem_s = (sem_sa, sem_sb)
    iota = lax.iota(jnp.int32, 16)
    perms = [(iota + i) & 15 for i in range(16)]
    rows16 = (iota, perms)

    def start(j, slot):
        pltpu.async_copy(table_hbm.at[idx_v.at[j]], bufs[slot], sem_g[slot])

    start(0, 0)
    start(1, 1)

    def loop(g, _):
        j = 2 * g
        for b in range(2):
            _process(idx_v, table_hbm, out_hbm, bufs, stages, sem_g, sem_s,
                     rows16, j + b, b, wid)
            start(j + b + 2, b)
        return 0

    lax.fori_loop(0, PER_W // 2 - 1, loop, 0)
    for b in range(2):
        _process(idx_v, table_hbm, out_hbm, bufs, stages, sem_g, sem_s,
                 rows16, PER_W - 2 + b, b, wid)
    for b in range(2):
        k2 = wid * PER_W + PER_W - 2 + b
        for tr in range(8):
            pltpu.make_async_copy(
                stages[b].at[pl.ds(8 * tr, 8), :],
                out_hbm.at[k2 // CHUNK, tr, lax.rem(k2, CHUNK)],
                sem_s[b],
            ).wait()


@jax.jit
def _gather(idx_grouped, emb_table):
    mesh = plsc.VectorSubcoreMesh(core_axis_name="c", subcore_axis_name="s")
    kfn = functools.partial(
        pl.kernel,
        mesh=mesh,
        out_type=jax.ShapeDtypeStruct((HIST, 8, 128, 8, 128), jnp.float32),
        scratch_types=(
            [pltpu.VMEM((PER_W, CHUNK), jnp.int32)]
            + [pltpu.VMEM((CHUNK, D), jnp.float32) for _ in range(2)]
            + [pltpu.VMEM((D, CHUNK), jnp.float32) for _ in range(2)]
            + [pltpu.SemaphoreType.DMA for _ in range(4)]
        ),
        compiler_params=pltpu.CompilerParams(
            use_tc_tiling_on_sc=False, needs_layout_passes=False
        ),
    )(_gather_body)
    return kfn(idx_grouped, emb_table)


def kernel(indices, emb_table):
    idx_grouped = indices.T.reshape(NW, PER_W, CHUNK).astype(jnp.int32)
    out5 = _gather(idx_grouped, emb_table)
    return out5.transpose(2, 4, 0, 1, 3).reshape(BATCH, HIST, D)


# speedup vs baseline: 1.4997x; 1.0459x over previous
import functools

import jax
import jax.numpy as jnp
from jax import lax
from jax.experimental import pallas as pl
from jax.experimental.pallas import tpu as pltpu
from jax.experimental.pallas import tpu_sc as plsc

VOCAB = 1000000
D = 64
BATCH = 16384
HIST = 50
NW = 32
CHUNK = 128
NBLK = HIST * (BATCH // CHUNK)
PER_W = NBLK // NW
BLK_BYTES = 8 * 8 * 128 * 4


def _process(idx_v, table_hbm, out_hbm, bufs, stages, sem_g, sem_s, rows16,
             jj, s, wid):
    k = wid * PER_W + jj
    h = k // CHUNK
    c = lax.rem(k, CHUNK)
    buf, stage = bufs[s], stages[s]
    pltpu.make_async_copy(table_hbm.at[idx_v.at[jj]], buf, sem_g[s]).wait()

    @pl.when(jj >= 2)
    def _():
        k2 = k - 2
        pltpu.make_async_copy(
            stage, out_hbm.at[k2 // CHUNK, :, lax.rem(k2, CHUNK)], sem_s[s]
        ).wait()

    iota, perms = rows16

    for dblk in range(4):
        cols = [perms[i] + 16 * dblk for i in range(16)]
        trs = [col >> 3 for col in cols]
        s8s = [col & 7 for col in cols]

        def tloop(g, _):
            rows = iota + 16 * g
            vals = [plsc.load_gather(buf, [rows, col]) for col in cols]
            for tr8, s8, v in zip(trs, s8s, vals):
                plsc.store_scatter(stage, [tr8, s8, rows], v)
            return 0

        lax.fori_loop(0, 8, tloop, 0)

    pltpu.async_copy(stage, out_hbm.at[h, :, c], sem_s[s])


def _gather_body(idx_hbm, table_hbm, out_hbm, idx_v, buf_a, buf_b,
                 stage_a, stage_b, sem_ga, sem_gb, sem_sa, sem_sb):
    wid = lax.axis_index("s") * 2 + lax.axis_index("c")
    pltpu.sync_copy(idx_hbm.at[wid], idx_v)

    bufs = (buf_a, buf_b)
    stages = (stage_a, stage_b)
    sem_g = (sem_ga, sem_gb)
    sem_s = (sem_sa, sem_sb)
    iota = lax.iota(jnp.int32, 16)
    perms = [(iota + i) & 15 for i in range(16)]
    rows16 = (iota, perms)

    def start(j, slot):
        pltpu.async_copy(table_hbm.at[idx_v.at[j]], bufs[slot], sem_g[slot])

    start(0, 0)
    start(1, 1)

    def loop(g, _):
        j = 2 * g
        for b in range(2):
            _process(idx_v, table_hbm, out_hbm, bufs, stages, sem_g, sem_s,
                     rows16, j + b, b, wid)
            start(j + b + 2, b)
        return 0

    lax.fori_loop(0, PER_W // 2 - 1, loop, 0)
    for b in range(2):
        _process(idx_v, table_hbm, out_hbm, bufs, stages, sem_g, sem_s,
                 rows16, PER_W - 2 + b, b, wid)
    for b in range(2):
        k2 = wid * PER_W + PER_W - 2 + b
        pltpu.make_async_copy(
            stages[b], out_hbm.at[k2 // CHUNK, :, lax.rem(k2, CHUNK)], sem_s[b]
        ).wait()


@jax.jit
def _gather(idx_grouped, emb_table):
    mesh = plsc.VectorSubcoreMesh(core_axis_name="c", subcore_axis_name="s")
    kfn = functools.partial(
        pl.kernel,
        mesh=mesh,
        out_type=jax.ShapeDtypeStruct((HIST, 8, 128, 8, 128), jnp.float32),
        scratch_types=(
            [pltpu.VMEM((PER_W, CHUNK), jnp.int32)]
            + [pltpu.VMEM((CHUNK, D), jnp.float32) for _ in range(2)]
            + [pltpu.VMEM((8, 8, CHUNK), jnp.float32) for _ in range(2)]
            + [pltpu.SemaphoreType.DMA for _ in range(4)]
        ),
        compiler_params=pltpu.CompilerParams(
            use_tc_tiling_on_sc=False, needs_layout_passes=False
        ),
    )(_gather_body)
    return kfn(idx_grouped, emb_table)


def kernel(indices, emb_table):
    idx_grouped = indices.T.reshape(NW, PER_W, CHUNK).astype(jnp.int32)
    out5 = _gather(idx_grouped, emb_table)
    return out5.transpose(2, 4, 0, 1, 3).reshape(BATCH, HIST, D)


# speedup vs baseline: 1.5703x; 1.0471x over previous
import functools

import jax
import jax.numpy as jnp
from jax import lax
from jax.experimental import pallas as pl
from jax.experimental.pallas import tpu as pltpu
from jax.experimental.pallas import tpu_sc as plsc

VOCAB = 1000000
D = 64
BATCH = 16384
HIST = 50
NW = 32
CHUNK = 128
NBLK = HIST * (BATCH // CHUNK)
PER_W = NBLK // NW
NBUF = 4
BLK_BYTES = 8 * 8 * 128 * 4


def _process(idx_v, table_hbm, out_hbm, bufs, stages, sem_g, sem_s, rows16,
             jj, s, wid):
    k = wid * PER_W + jj
    h = k // CHUNK
    c = lax.rem(k, CHUNK)
    buf, stage = bufs[s], stages[s]
    pltpu.make_async_copy(table_hbm.at[idx_v.at[jj]], buf, sem_g[s]).wait()

    @pl.when(jj >= NBUF)
    def _():
        k2 = k - NBUF
        pltpu.make_async_copy(
            stage, out_hbm.at[k2 // CHUNK, :, lax.rem(k2, CHUNK)], sem_s[s]
        ).wait()

    iota, perms = rows16

    for dblk in range(4):
        cols = [perms[i] + 16 * dblk for i in range(16)]
        trs = [col >> 3 for col in cols]
        s8s = [col & 7 for col in cols]

        def tloop(g, _):
            rows = iota + 16 * g
            vals = [plsc.load_gather(buf, [rows, col]) for col in cols]
            for tr8, s8, v in zip(trs, s8s, vals):
                plsc.store_scatter(stage, [tr8, s8, rows], v)
            return 0

        lax.fori_loop(0, 8, tloop, 0)

    pltpu.async_copy(stage, out_hbm.at[h, :, c], sem_s[s])


def _gather_body(idx_hbm, table_hbm, out_hbm, idx_v, *scratch):
    wid = lax.axis_index("s") * 2 + lax.axis_index("c")
    pltpu.sync_copy(idx_hbm.at[wid], idx_v)

    bufs = scratch[:NBUF]
    stages = scratch[NBUF:2 * NBUF]
    sem_g = scratch[2 * NBUF:3 * NBUF]
    sem_s = scratch[3 * NBUF:]
    iota = lax.iota(jnp.int32, 16)
    perms = [(iota + i) & 15 for i in range(16)]
    rows16 = (iota, perms)

    def start(j, slot):
        pltpu.async_copy(table_hbm.at[idx_v.at[j]], bufs[slot], sem_g[slot])

    for b in range(NBUF):
        start(b, b)

    def loop(g, _):
        j = NBUF * g
        for b in range(NBUF):
            _process(idx_v, table_hbm, out_hbm, bufs, stages, sem_g, sem_s,
                     rows16, j + b, b, wid)
            start(j + b + NBUF, b)
        return 0

    lax.fori_loop(0, PER_W // NBUF - 1, loop, 0)
    for b in range(NBUF):
        _process(idx_v, table_hbm, out_hbm, bufs, stages, sem_g, sem_s,
                 rows16, PER_W - NBUF + b, b, wid)
    for b in range(NBUF):
        k2 = wid * PER_W + PER_W - NBUF + b
        pltpu.make_async_copy(
            stages[b], out_hbm.at[k2 // CHUNK, :, lax.rem(k2, CHUNK)], sem_s[b]
        ).wait()


@jax.jit
def _gather(idx_grouped, emb_table):
    mesh = plsc.VectorSubcoreMesh(core_axis_name="c", subcore_axis_name="s")
    kfn = functools.partial(
        pl.kernel,
        mesh=mesh,
        out_type=jax.ShapeDtypeStruct((HIST, 8, 128, 8, 128), jnp.float32),
        scratch_types=(
            [pltpu.VMEM((PER_W, CHUNK), jnp.int32)]
            + [pltpu.VMEM((CHUNK, D), jnp.float32) for _ in range(NBUF)]
            + [pltpu.VMEM((8, 8, CHUNK), jnp.float32) for _ in range(NBUF)]
            + [pltpu.SemaphoreType.DMA for _ in range(2 * NBUF)]
        ),
        compiler_params=pltpu.CompilerParams(
            use_tc_tiling_on_sc=False, needs_layout_passes=False
        ),
    )(_gather_body)
    return kfn(idx_grouped, emb_table)


def kernel(indices, emb_table):
    idx_grouped = indices.T.reshape(NW, PER_W, CHUNK).astype(jnp.int32)
    out5 = _gather(idx_grouped, emb_table)
    return out5.transpose(2, 4, 0, 1, 3).reshape(BATCH, HIST, D)
